# Initial kernel scaffold; baseline (speedup 1.0000x reference)
import functools

import jax
import jax.numpy as jnp
from jax import lax
from jax.experimental import pallas as pl
from jax.experimental.pallas import tpu as pltpu
from jax.experimental.pallas import tpu_sc as plsc

BATCH = 16384
VOCAB = 100000
EMB = 128
CTX = 6
OUT = 91

ROWS = BATCH * CTX
NUM_WORKERS = 32
ROWS_PER_W = ROWS // NUM_WORKERS
CHUNK = 512
NCHUNK = ROWS_PER_W // CHUNK


def _gather_body(table_hbm, idx_hbm, out_hbm, idx_v, rows_a, rows_b, sem_a, sem_b):
    wid = lax.axis_index("s") * 2 + lax.axis_index("c")
    base = wid * ROWS_PER_W
    pltpu.sync_copy(idx_hbm.at[pl.ds(base, ROWS_PER_W)], idx_v)
    bufs = (rows_a, rows_b)
    sems = (sem_a, sem_b)
    cps = [None, None]
    cps[0] = pltpu.async_copy(table_hbm.at[idx_v.at[pl.ds(0, CHUNK)]], bufs[0], sems[0])
    for c in range(NCHUNK):
        nxt = (c + 1) % 2
        if c + 1 < NCHUNK:
            cps[nxt] = pltpu.async_copy(
                table_hbm.at[idx_v.at[pl.ds((c + 1) * CHUNK, CHUNK)]], bufs[nxt], sems[nxt])
        cps[c % 2].wait()
        pltpu.sync_copy(bufs[c % 2], out_hbm.at[pl.ds(base + c * CHUNK, CHUNK)])


_gather = pl.kernel(
    _gather_body,
    out_type=jax.ShapeDtypeStruct((ROWS, EMB), jnp.float32),
    mesh=plsc.VectorSubcoreMesh(core_axis_name="c", subcore_axis_name="s"),
    scratch_types=[
        pltpu.VMEM((ROWS_PER_W,), jnp.int32),
        pltpu.VMEM((CHUNK, EMB), jnp.float32),
        pltpu.VMEM((CHUNK, EMB), jnp.float32),
        pltpu.SemaphoreType.DMA,
        pltpu.SemaphoreType.DMA,
    ],
)


def _mlp_body(x_ref, w1_ref, b1_ref, w2_ref, b2_ref, out_ref):
    x = x_ref[...]
    h = jnp.maximum(
        jax.lax.dot_general(x, w1_ref[...], (((1,), (0,)), ((), ())),
                            preferred_element_type=jnp.float32) + b1_ref[...],
        0.0)
    logits = jax.lax.dot_general(h, w2_ref[...], (((1,), (0,)), ((), ())),
                                 preferred_element_type=jnp.float32) + b2_ref[...]
    m = jnp.max(logits, axis=1, keepdims=True)
    s = logits - m
    lse = jnp.log(jnp.sum(jnp.exp(s), axis=1, keepdims=True))
    out_ref[...] = s - lse


BLOCK_B = 1024


def _mlp(x, W1, b1, W2, b2):
    grid = (BATCH // BLOCK_B,)
    return pl.pallas_call(
        _mlp_body,
        grid=grid,
        in_specs=[
            pl.BlockSpec((BLOCK_B, CTX * EMB), lambda i: (i, 0)),
            pl.BlockSpec((CTX * EMB, EMB), lambda i: (0, 0)),
            pl.BlockSpec((1, EMB), lambda i: (0, 0)),
            pl.BlockSpec((EMB, OUT), lambda i: (0, 0)),
            pl.BlockSpec((1, OUT), lambda i: (0, 0)),
        ],
        out_specs=pl.BlockSpec((BLOCK_B, OUT), lambda i: (i, 0)),
        out_shape=jax.ShapeDtypeStruct((BATCH, OUT), jnp.float32),
    )(x, W1, b1, W2, b2)


@jax.jit
def kernel(inputs, emb, W1, b1, W2, b2):
    idx = inputs.reshape(-1)
    gathered = _gather(emb, idx)
    x = gathered.reshape(BATCH, CTX * EMB)
    return _mlp(x, W1, b1.reshape(1, EMB), W2, b2.reshape(1, OUT))


# speedup vs baseline: 3.3235x; 3.3235x over previous
import functools

import jax
import jax.numpy as jnp
from jax import lax
from jax.experimental import pallas as pl
from jax.experimental.pallas import tpu as pltpu
from jax.experimental.pallas import tpu_sc as plsc

BATCH = 16384
VOCAB = 100000
EMB = 128
CTX = 6
OUT = 91

ROWS = BATCH * CTX
NUM_WORKERS = 32
ROWS_PER_W = ROWS // NUM_WORKERS
CHUNK = 384
NCHUNK = ROWS_PER_W // CHUNK


def _gather_body(table_hbm, idx_hbm, out_hbm, idx_v, rows_a, rows_b, sem_a, sem_b):
    wid = lax.axis_index("s") * 2 + lax.axis_index("c")
    base = wid * ROWS_PER_W
    pltpu.sync_copy(idx_hbm.at[pl.ds(base, ROWS_PER_W)], idx_v)
    bufs = (rows_a, rows_b)
    sems = (sem_a, sem_b)
    cps = [None, None]
    cps[0] = pltpu.async_copy(table_hbm.at[idx_v.at[pl.ds(0, CHUNK)]], bufs[0], sems[0])
    for c in range(NCHUNK):
        nxt = (c + 1) % 2
        if c + 1 < NCHUNK:
            cps[nxt] = pltpu.async_copy(
                table_hbm.at[idx_v.at[pl.ds((c + 1) * CHUNK, CHUNK)]], bufs[nxt], sems[nxt])
        cps[c % 2].wait()
        pltpu.sync_copy(bufs[c % 2], out_hbm.at[pl.ds(base + c * CHUNK, CHUNK)])


_gather = pl.kernel(
    _gather_body,
    out_type=jax.ShapeDtypeStruct((ROWS, EMB), jnp.float32),
    mesh=plsc.VectorSubcoreMesh(core_axis_name="c", subcore_axis_name="s"),
    scratch_types=[
        pltpu.VMEM((ROWS_PER_W,), jnp.int32),
        pltpu.VMEM((CHUNK, EMB), jnp.float32),
        pltpu.VMEM((CHUNK, EMB), jnp.float32),
        pltpu.SemaphoreType.DMA,
        pltpu.SemaphoreType.DMA,
    ],
)


def _mlp_body(x_ref, w1_ref, b1_ref, w2_ref, b2_ref, out_ref):
    x = x_ref[...]
    h = jnp.maximum(
        jax.lax.dot_general(x, w1_ref[...], (((1,), (0,)), ((), ())),
                            preferred_element_type=jnp.float32) + b1_ref[...],
        0.0)
    logits = jax.lax.dot_general(h, w2_ref[...], (((1,), (0,)), ((), ())),
                                 preferred_element_type=jnp.float32) + b2_ref[...]
    m = jnp.max(logits, axis=1, keepdims=True)
    s = logits - m
    lse = jnp.log(jnp.sum(jnp.exp(s), axis=1, keepdims=True))
    out_ref[...] = s - lse


BLOCK_B = 1024


def _mlp(x, W1, b1, W2, b2):
    grid = (BATCH // BLOCK_B,)
    return pl.pallas_call(
        _mlp_body,
        grid=grid,
        in_specs=[
            pl.BlockSpec((BLOCK_B, CTX * EMB), lambda i: (i, 0)),
            pl.BlockSpec((CTX * EMB, EMB), lambda i: (0, 0)),
            pl.BlockSpec((1, EMB), lambda i: (0, 0)),
            pl.BlockSpec((EMB, OUT), lambda i: (0, 0)),
            pl.BlockSpec((1, OUT), lambda i: (0, 0)),
        ],
        out_specs=pl.BlockSpec((BLOCK_B, OUT), lambda i: (i, 0)),
        out_shape=jax.ShapeDtypeStruct((BATCH, OUT), jnp.float32),
    )(x, W1, b1, W2, b2)


@jax.jit
def kernel(inputs, emb, W1, b1, W2, b2):
    idx = inputs.reshape(-1)
    gathered = _gather(emb, idx)
    x = gathered.reshape(BATCH, CTX * EMB)
    return _mlp(x, W1, b1.reshape(1, EMB), W2, b2.reshape(1, OUT))
